# baseline (speedup 1.0000x reference)
import functools

import jax
import jax.numpy as jnp
from jax.experimental import pallas as pl
from jax.experimental.pallas import tpu as pltpu
from jax.experimental.pallas import tpu_sc as plsc

_NG = 128
_GS = 32
_B = 16
_N = 8192
_SL = 64
_LN = 128
_TOT = _B * _NG * _GS

_NC = 2
_NS = 16
_NW = _NC * _NS
_BPW = _TOT // _NW
_RPB = _NG * _GS
_VL = 16
_NV = _BPW // _VL


def _fps_kernel(xs_ref, ys_ref, zs_ref, cent_ref, dist_scr):
    xs = xs_ref[...]
    ys = ys_ref[...]
    zs = zs_ref[...]
    si = jax.lax.broadcasted_iota(jnp.int32, (1, _SL, _LN), 1)
    li = jax.lax.broadcasted_iota(jnp.int32, (1, _SL, _LN), 2)
    fi = si * _LN + li
    dist_scr[...] = jnp.full((_B, _SL, _LN), 1e10, jnp.float32)

    def body(i, far):
        mask = fi == far
        cx = jnp.sum(jnp.where(mask, xs, 0.0), axis=(1, 2), keepdims=True)
        cy = jnp.sum(jnp.where(mask, ys, 0.0), axis=(1, 2), keepdims=True)
        cz = jnp.sum(jnp.where(mask, zs, 0.0), axis=(1, 2), keepdims=True)
        row = jnp.concatenate([cx[:, 0, :], cy[:, 0, :], cz[:, 0, :]],
                              axis=-1)
        cent_ref[:, pl.ds(i, 1), :] = row[:, None, :]
        dx = xs - cx
        dy = ys - cy
        dz = zs - cz
        d = (dx * dx + dy * dy) + dz * dz
        dmin = jnp.minimum(dist_scr[...], d)
        dist_scr[...] = dmin
        m = jnp.max(dmin, axis=(1, 2), keepdims=True)
        far2 = jnp.min(jnp.where(dmin == m, fi, _N), axis=(1, 2),
                       keepdims=True)
        return far2

    jax.lax.fori_loop(0, _NG, body, jnp.zeros((_B, 1, 1), jnp.int32))


def _knn_kernel(xs_ref, ys_ref, zs_ref, cent_ref, idx_ref, gidx_ref, d_scr):
    px = xs_ref[0]
    py = ys_ref[0]
    pz = zs_ref[0]
    cg = cent_ref[0]
    cgx = cg[:, 0:1]
    cgy = cg[:, 1:2]
    cgz = cg[:, 2:3]
    aa = (cgx * cgx + cgy * cgy) + cgz * cgz
    bb = (px * px + py * py) + pz * pz
    cbx = cgx.astype(jnp.bfloat16).astype(jnp.float32)
    cby = cgy.astype(jnp.bfloat16).astype(jnp.float32)
    cbz = cgz.astype(jnp.bfloat16).astype(jnp.float32)
    pbx = px.astype(jnp.bfloat16).astype(jnp.float32)
    pby = py.astype(jnp.bfloat16).astype(jnp.float32)
    pbz = pz.astype(jnp.bfloat16).astype(jnp.float32)
    ab = (cbx * pbx + cby * pby) + cbz * pbz
    d2 = jnp.maximum(aa + bb - 2.0 * ab, 0.0)
    d_scr[...] = jnp.sqrt(d2)
    li = jax.lax.broadcasted_iota(jnp.int32, (1, _N), 1)
    pid = pl.program_id(0)
    for k in range(_GS):
        dcur = d_scr[...]
        m = jnp.min(dcur, axis=1, keepdims=True)
        a = jnp.min(jnp.where(dcur == m, li, _N), axis=1, keepdims=True)
        idx_ref[0, :, k:k + 1] = a
        gidx_ref[0, :, k:k + 1] = a + pid * _N
        d_scr[...] = jnp.where(li == a, jnp.inf, dcur)


def _make_sc_gather():
    mesh = plsc.VectorSubcoreMesh(core_axis_name="c", subcore_axis_name="s")
    f32 = jnp.float32

    @functools.partial(
        pl.kernel, mesh=mesh,
        out_type=(
            jax.ShapeDtypeStruct((_TOT,), f32),
            jax.ShapeDtypeStruct((_TOT,), f32),
            jax.ShapeDtypeStruct((_TOT,), f32),
        ),
        scratch_types=[
            pltpu.VMEM((_BPW,), jnp.int32),
            pltpu.VMEM((_BPW,), jnp.int32),
            pltpu.VMEM((_BPW,), f32),
            pltpu.VMEM((_BPW,), f32),
            pltpu.VMEM((_BPW,), f32),
            pltpu.VMEM((_BPW,), f32),
            pltpu.VMEM((_BPW,), f32),
            pltpu.VMEM((_BPW,), f32),
        ],
    )
    def gather_k(xs_hbm, ys_hbm, zs_hbm, cx_hbm, cy_hbm, cz_hbm, gidx_hbm,
                 cidx_hbm, ox_hbm, oy_hbm, oz_hbm,
                 idx_v, cidx_v, gx_v, gy_v, gz_v, cx_v, cy_v, cz_v):
        wid = jax.lax.axis_index("s") * _NC + jax.lax.axis_index("c")
        base = wid * _BPW
        pltpu.sync_copy(gidx_hbm.at[pl.ds(base, _BPW)], idx_v)
        pltpu.sync_copy(cidx_hbm.at[pl.ds(base, _BPW)], cidx_v)
        pltpu.sync_copy(xs_hbm.at[idx_v], gx_v)
        pltpu.sync_copy(ys_hbm.at[idx_v], gy_v)
        pltpu.sync_copy(zs_hbm.at[idx_v], gz_v)
        pltpu.sync_copy(cx_hbm.at[cidx_v], cx_v)
        pltpu.sync_copy(cy_hbm.at[cidx_v], cy_v)
        pltpu.sync_copy(cz_hbm.at[cidx_v], cz_v)

        def body(i, carry):
            sl = pl.ds(i * _VL, _VL)
            gx_v[sl] = gx_v[sl] - cx_v[sl]
            gy_v[sl] = gy_v[sl] - cy_v[sl]
            gz_v[sl] = gz_v[sl] - cz_v[sl]
            return carry

        jax.lax.fori_loop(0, _NV, body, 0)
        pltpu.sync_copy(gx_v, ox_hbm.at[pl.ds(base, _BPW)])
        pltpu.sync_copy(gy_v, oy_hbm.at[pl.ds(base, _BPW)])
        pltpu.sync_copy(gz_v, oz_hbm.at[pl.ds(base, _BPW)])

    return gather_k


_sc_gather = _make_sc_gather()


def kernel(xyz):
    B, N, _ = xyz.shape
    xs = xyz[:, :, 0]
    ys = xyz[:, :, 1]
    zs = xyz[:, :, 2]
    xs3 = xs.reshape(B, _SL, _LN)
    ys3 = ys.reshape(B, _SL, _LN)
    zs3 = zs.reshape(B, _SL, _LN)

    centers = pl.pallas_call(
        _fps_kernel,
        out_shape=jax.ShapeDtypeStruct((B, _NG, 3), jnp.float32),
        scratch_shapes=[pltpu.VMEM((_B, _SL, _LN), jnp.float32)],
    )(xs3, ys3, zs3)

    idx = pl.pallas_call(
        _knn_kernel,
        grid=(B,),
        in_specs=[
            pl.BlockSpec((1, 1, N), lambda b: (b, 0, 0)),
            pl.BlockSpec((1, 1, N), lambda b: (b, 0, 0)),
            pl.BlockSpec((1, 1, N), lambda b: (b, 0, 0)),
            pl.BlockSpec((1, _NG, 3), lambda b: (b, 0, 0)),
        ],
        out_specs=[
            pl.BlockSpec((1, _NG, _GS), lambda b: (b, 0, 0)),
            pl.BlockSpec((1, _NG, _GS), lambda b: (b, 0, 0)),
        ],
        out_shape=[
            jax.ShapeDtypeStruct((B, _NG, _GS), jnp.int32),
            jax.ShapeDtypeStruct((B, _NG, _GS), jnp.int32),
        ],
        scratch_shapes=[pltpu.VMEM((_NG, _N), jnp.float32)],
    )(xs[:, None, :], ys[:, None, :], zs[:, None, :], centers)
    idx, gidx = idx

    cidx = (jnp.arange(_TOT, dtype=jnp.int32) // _GS)
    ox, oy, oz = _sc_gather(
        xs.reshape(B * N), ys.reshape(B * N), zs.reshape(B * N),
        centers[:, :, 0].reshape(B * _NG), centers[:, :, 1].reshape(B * _NG),
        centers[:, :, 2].reshape(B * _NG),
        gidx.reshape(_TOT), cidx,
    )
    neighborhood = jnp.stack(
        [ox.reshape(B, _NG, _GS), oy.reshape(B, _NG, _GS),
         oz.reshape(B, _NG, _GS)], axis=-1)
    return neighborhood, centers, idx
